# baseline (speedup 1.0000x reference)
import jax
import jax.numpy as jnp
from jax.experimental import pallas as pl

_N = 10000
_C = 128
_BLK = 2000


def _body(x_ref, wxz_ref, wxh_ref, wlin_ref,
          bxz_ref, bhz_ref, bxh_ref, bhh_ref, blin_ref, o_ref):
    xb = x_ref[...]
    a = jnp.dot(xb, wxz_ref[...], preferred_element_type=jnp.float32)
    a = a + (bxz_ref[...] + bhz_ref[...])
    b = jnp.dot(xb, wxh_ref[...], preferred_element_type=jnp.float32)
    b = b + (bxh_ref[...] + bhh_ref[...])
    hpre = jax.nn.sigmoid(-a) * jnp.tanh(b)
    h = jnp.where(hpre > 0, hpre, jnp.exp(hpre) - 1.0)
    o_ref[...] = (
        jnp.dot(h, wlin_ref[...], preferred_element_type=jnp.float32)
        + blin_ref[...]
    )


def kernel(x, edge_index, edge_weight, W_xz, b_xz, W_hz, b_hz, W_xr, b_xr,
           W_hr, b_hr, W_xh, b_xh, W_hh, b_hh, W_lin, b_lin):
    grid = (_N // _BLK,)
    full = lambda i: (0, 0)
    wspec = pl.BlockSpec((_C, _C), full)
    bspec = pl.BlockSpec((1, _C), full)
    return pl.pallas_call(
        _body,
        grid=grid,
        in_specs=[
            pl.BlockSpec((_BLK, _C), lambda i: (i, 0)),
            wspec, wspec, wspec,
            bspec, bspec, bspec, bspec, bspec,
        ],
        out_specs=pl.BlockSpec((_BLK, _C), lambda i: (i, 0)),
        out_shape=jax.ShapeDtypeStruct((_N, _C), jnp.float32),
    )(x, W_xz, W_xh, W_lin,
      b_xz.reshape(1, _C), b_hz.reshape(1, _C),
      b_xh.reshape(1, _C), b_hh.reshape(1, _C), b_lin.reshape(1, _C))


# speedup vs baseline: 1.2438x; 1.2438x over previous
import jax
import jax.numpy as jnp
from jax.experimental import pallas as pl

_N = 10000
_C = 128
_BLK = 5000


def _body(x_ref, wxz_ref, wxh_ref, wlin_ref,
          bxz_ref, bhz_ref, bxh_ref, bhh_ref, blin_ref, o_ref):
    xb = x_ref[...]
    wcat = jnp.concatenate([wxz_ref[...], wxh_ref[...]], axis=1)
    t = jnp.dot(xb, wcat, preferred_element_type=jnp.float32)
    a = t[:, :_C] + (bxz_ref[...] + bhz_ref[...])
    b = t[:, _C:] + (bxh_ref[...] + bhh_ref[...])
    hpre = jax.nn.sigmoid(-a) * jnp.tanh(b)
    h = jnp.where(hpre > 0, hpre, jnp.exp(hpre) - 1.0)
    o_ref[...] = (
        jnp.dot(h, wlin_ref[...], preferred_element_type=jnp.float32)
        + blin_ref[...]
    )


def kernel(x, edge_index, edge_weight, W_xz, b_xz, W_hz, b_hz, W_xr, b_xr,
           W_hr, b_hr, W_xh, b_xh, W_hh, b_hh, W_lin, b_lin):
    grid = (_N // _BLK,)
    full = lambda i: (0, 0)
    wspec = pl.BlockSpec((_C, _C), full)
    bspec = pl.BlockSpec((1, _C), full)
    return pl.pallas_call(
        _body,
        grid=grid,
        in_specs=[
            pl.BlockSpec((_BLK, _C), lambda i: (i, 0)),
            wspec, wspec, wspec,
            bspec, bspec, bspec, bspec, bspec,
        ],
        out_specs=pl.BlockSpec((_BLK, _C), lambda i: (i, 0)),
        out_shape=jax.ShapeDtypeStruct((_N, _C), jnp.float32),
    )(x, W_xz, W_xh, W_lin,
      b_xz.reshape(1, _C), b_hz.reshape(1, _C),
      b_xh.reshape(1, _C), b_hh.reshape(1, _C), b_lin.reshape(1, _C))
